# baseline (speedup 1.0000x reference)
import functools

import jax
import jax.numpy as jnp
from jax import lax
from jax.experimental import pallas as pl
from jax.experimental.pallas import tpu as pltpu
from jax.experimental.pallas import tpu_sc as plsc

N = 10000
E = 320000
IN_DIM = 128
H = 8
D = 16
HD = H * D
ROW = HD + D

NC = 2
NS = 16
NW = NC * NS
EPW = E // NW
N_ACC = 10240
RPT = N_ACC // NS
B = 64
NB = EPW // B
TAIL = EPW - NB * B


def _proj_body(h_ref, wq_ref, wkv_ref, q_ref, kv_ref):
    hb = h_ref[...]
    q_ref[...] = jnp.dot(hb, wq_ref[...], preferred_element_type=jnp.float32)
    kv_ref[...] = jnp.dot(hb, wkv_ref[...], preferred_element_type=jnp.float32)


def _project(h, wq, wkv):
    blk = 1000
    grid = (N // blk,)
    return pl.pallas_call(
        _proj_body,
        grid=grid,
        in_specs=[
            pl.BlockSpec((blk, IN_DIM), lambda i: (i, 0)),
            pl.BlockSpec((IN_DIM, HD), lambda i: (0, 0)),
            pl.BlockSpec((IN_DIM, 2 * HD), lambda i: (0, 0)),
        ],
        out_specs=[
            pl.BlockSpec((blk, HD), lambda i: (i, 0)),
            pl.BlockSpec((blk, 2 * HD), lambda i: (i, 0)),
        ],
        out_shape=[
            jax.ShapeDtypeStruct((N, HD), jnp.float32),
            jax.ShapeDtypeStruct((N, 2 * HD), jnp.float32),
        ],
    )(h, wq, wkv)


def _edge_body(q_hbm, kv_hbm, src_hbm, dst_hbm, partial_hbm,
               src_v, dst_v, src_t, dst_t, kv_rows, q_rows, out_rows,
               acc, sem1, sem2, sem3, sem4):
    cid = lax.axis_index("c")
    sid = lax.axis_index("s")
    wid = sid * NC + cid
    base = wid * EPW

    lane = lax.iota(jnp.int32, 16)

    def _zo(i, carry):
        r = i // (ROW // 16)
        c = i % (ROW // 16)
        out_rows[r, pl.ds(c * 16, 16)] = jnp.zeros((16,), jnp.float32)
        return carry

    lax.fori_loop(0, B * (ROW // 16), _zo, 0)
    for j in range(RPT // B):
        pltpu.sync_copy(out_rows, acc.at[pl.ds(sid * RPT + j * B, B)])
    plsc.subcore_barrier()

    inv_sqrt_d = 0.25

    def _group16(rows):
        def _sbody(tt, parts):
            t = tt * 4
            out = list(parts)
            for k in range(4):
                rot = (lane + (t + k)) & (D - 1)
                for h in range(H):
                    col = rot + (h * D)
                    kk = plsc.load_gather(kv_rows, [rows, col])
                    qq = plsc.load_gather(q_rows, [rows, col])
                    out[h] = out[h] + kk * qq
            return tuple(out)

        parts = plsc.parallel_loop(
            0, D // 4, carry=tuple(jnp.zeros((16,), jnp.float32)
                                   for _ in range(H)))(_sbody)
        ps = [jnp.exp(jnp.clip(s * inv_sqrt_d, -5.0, 5.0)) for s in parts]

        def _vbody(tt):
            t = tt * 4
            for k in range(4):
                rot = (lane + (t + k)) & (D - 1)
                for h in range(H):
                    ocol = rot + (h * D)
                    vv = plsc.load_gather(kv_rows, [rows, ocol + HD])
                    plsc.store_scatter(out_rows, [rows, ocol], vv * ps[h])

        plsc.parallel_loop(0, D // 4)(_vbody)
        for h in range(H):
            dcol = jnp.full((16,), HD + h, jnp.int32)
            plsc.store_scatter(out_rows, [rows, dcol], ps[h])

    def _group(g):
        _group16(g * 16 + lane)

    def _batch(i, carry):
        eb = base + i * B
        hb = B // 2
        pltpu.sync_copy(src_hbm.at[pl.ds(eb, B)], src_v)
        pltpu.sync_copy(dst_hbm.at[pl.ds(eb, B)], dst_v)
        c1a = pltpu.async_copy(kv_hbm.at[src_v.at[pl.ds(0, hb)]],
                               kv_rows.at[pl.ds(0, hb)], sem1)
        c2a = pltpu.async_copy(q_hbm.at[dst_v.at[pl.ds(0, hb)]],
                               q_rows.at[pl.ds(0, hb)], sem2)
        c1b = pltpu.async_copy(kv_hbm.at[src_v.at[pl.ds(hb, hb)]],
                               kv_rows.at[pl.ds(hb, hb)], sem3)
        c2b = pltpu.async_copy(q_hbm.at[dst_v.at[pl.ds(hb, hb)]],
                               q_rows.at[pl.ds(hb, hb)], sem4)
        c1a.wait()
        c2a.wait()
        plsc.parallel_loop(0, B // 32)(_group)
        c1b.wait()
        c2b.wait()
        plsc.parallel_loop(B // 32, B // 16)(_group)
        pltpu.sync_copy(out_rows, acc.at[dst_v], add=True)
        return carry

    lax.fori_loop(0, NB, _batch, 0)

    et = base + NB * B
    pltpu.sync_copy(src_hbm.at[pl.ds(et, TAIL)], src_t)
    pltpu.sync_copy(dst_hbm.at[pl.ds(et, TAIL)], dst_t)
    c1 = pltpu.async_copy(kv_hbm.at[src_t], kv_rows.at[pl.ds(0, TAIL)], sem1)
    c2 = pltpu.async_copy(q_hbm.at[dst_t], q_rows.at[pl.ds(0, TAIL)], sem2)
    c1.wait()
    c2.wait()
    _group16(lane)
    pltpu.sync_copy(out_rows.at[pl.ds(0, TAIL)], acc.at[dst_t], add=True)

    plsc.subcore_barrier()
    pltpu.sync_copy(acc.at[pl.ds(sid * RPT, RPT)],
                    partial_hbm.at[cid, pl.ds(sid * RPT, RPT)])


def _edge_pass(q, kv, src, dst):
    mesh = plsc.VectorSubcoreMesh(core_axis_name="c", subcore_axis_name="s",
                                  num_cores=NC, num_subcores=NS)
    f = pl.kernel(
        _edge_body,
        out_type=jax.ShapeDtypeStruct((NC, N_ACC, ROW), jnp.float32),
        mesh=mesh,
        scratch_types=[
            pltpu.VMEM((B,), jnp.int32),
            pltpu.VMEM((B,), jnp.int32),
            pltpu.VMEM((TAIL,), jnp.int32),
            pltpu.VMEM((TAIL,), jnp.int32),
            pltpu.VMEM((B, 2 * HD), jnp.float32),
            pltpu.VMEM((B, HD), jnp.float32),
            pltpu.VMEM((B, ROW), jnp.float32),
            pltpu.VMEM_SHARED((N_ACC, ROW), jnp.float32),
            pltpu.SemaphoreType.DMA,
            pltpu.SemaphoreType.DMA,
            pltpu.SemaphoreType.DMA,
            pltpu.SemaphoreType.DMA,
        ],
        compiler_params=pltpu.CompilerParams(use_tc_tiling_on_sc=False,
                                             needs_layout_passes=False),
    )
    return f(q, kv, src, dst)


def _final_body(p_ref, o_ref):
    s = p_ref[0] + p_ref[1]
    den = s[:, HD:HD + H]
    r = jnp.where(den > 0.0, 1.0 / den, 0.0)
    for h in range(H):
        o_ref[:, h * D:(h + 1) * D] = s[:, h * D:(h + 1) * D] * r[:, h:h + 1]


def _finalize(partial):
    blk = 1000
    return pl.pallas_call(
        _final_body,
        grid=(N // blk,),
        in_specs=[pl.BlockSpec((NC, blk, ROW), lambda i: (0, i, 0))],
        out_specs=pl.BlockSpec((blk, HD), lambda i: (i, 0)),
        out_shape=jax.ShapeDtypeStruct((N, HD), jnp.float32),
    )(partial)


def kernel(h, edge_index, Wq, Wk, Wv):
    wkv = jnp.concatenate([Wk, Wv], axis=1)
    q, kv = _project(h, Wq, wkv)
    partial = _edge_pass(q, kv, edge_index[0], edge_index[1])
    return _finalize(partial)


# speedup vs baseline: 1.0775x; 1.0775x over previous
import functools

import jax
import jax.numpy as jnp
from jax import lax
from jax.experimental import pallas as pl
from jax.experimental.pallas import tpu as pltpu
from jax.experimental.pallas import tpu_sc as plsc

N = 10000
E = 320000
IN_DIM = 128
H = 8
D = 16
HD = H * D
ROW = HD + H

NC = 2
NS = 16
NW = NC * NS
EPW = E // NW
N_ACC = 10240
RPT = N_ACC // NS
B = 80
NB = EPW // B


def _proj_body(h_ref, wq_ref, wkv_ref, q_ref, kv_ref):
    hb = h_ref[...]
    q_ref[...] = jnp.dot(hb, wq_ref[...], preferred_element_type=jnp.float32)
    kv_ref[...] = jnp.dot(hb, wkv_ref[...], preferred_element_type=jnp.float32)


def _project(h, wq, wkv):
    blk = 1000
    grid = (N // blk,)
    return pl.pallas_call(
        _proj_body,
        grid=grid,
        in_specs=[
            pl.BlockSpec((blk, IN_DIM), lambda i: (i, 0)),
            pl.BlockSpec((IN_DIM, HD), lambda i: (0, 0)),
            pl.BlockSpec((IN_DIM, 2 * HD), lambda i: (0, 0)),
        ],
        out_specs=[
            pl.BlockSpec((blk, HD), lambda i: (i, 0)),
            pl.BlockSpec((blk, 2 * HD), lambda i: (i, 0)),
        ],
        out_shape=[
            jax.ShapeDtypeStruct((N, HD), jnp.float32),
            jax.ShapeDtypeStruct((N, 2 * HD), jnp.float32),
        ],
    )(h, wq, wkv)


def _edge_body(q_hbm, kv_hbm, idx_hbm, partial_hbm,
               idx2, kv_rows, q_rows, out_rows,
               acc, sem1, sem2):
    cid = lax.axis_index("c")
    sid = lax.axis_index("s")
    wid = sid * NC + cid
    base_b = wid * NB

    lane = lax.iota(jnp.int32, 16)

    NZC = (ROW + 15) // 16
    def _zo(i, carry):
        r = i // NZC
        c = i % NZC
        col = jnp.minimum(c * 16, ROW - 16)
        out_rows[r, pl.ds(col, 16)] = jnp.zeros((16,), jnp.float32)
        return carry

    lax.fori_loop(0, B * NZC, _zo, 0)
    for j in range(RPT // B):
        pltpu.sync_copy(out_rows, acc.at[pl.ds(sid * RPT + j * B, B)])
    plsc.subcore_barrier()

    inv_sqrt_d = 0.25

    def _group16(rows):
        def _sbody(tt, parts):
            t = tt * 4
            out = list(parts)
            for k in range(4):
                rot = (lane + (t + k)) & (D - 1)
                for h in range(H):
                    col = rot + (h * D)
                    kk = plsc.load_gather(kv_rows, [rows, col])
                    qq = plsc.load_gather(q_rows, [rows, col])
                    out[h] = out[h] + kk * qq
            return tuple(out)

        parts = plsc.parallel_loop(
            0, D // 4, carry=tuple(jnp.zeros((16,), jnp.float32)
                                   for _ in range(H)))(_sbody)
        ps = [jnp.exp(jnp.clip(s * inv_sqrt_d, -5.0, 5.0)) for s in parts]

        def _vbody(tt):
            t = tt * 4
            for k in range(4):
                rot = (lane + (t + k)) & (D - 1)
                for h in range(H):
                    ocol = rot + (h * D)
                    vv = plsc.load_gather(kv_rows, [rows, ocol + HD])
                    plsc.store_scatter(out_rows, [rows, ocol], vv * ps[h])

        plsc.parallel_loop(0, D // 4)(_vbody)
        for h in range(H):
            dcol = jnp.full((16,), HD + h, jnp.int32)
            plsc.store_scatter(out_rows, [rows, dcol], ps[h])

    def _group(g):
        _group16(g * 16 + lane)

    def _batch(i, carry):
        pltpu.sync_copy(idx_hbm.at[base_b + i], idx2)
        src_r = idx2.at[0]
        dst_r = idx2.at[1]
        c1 = pltpu.async_copy(kv_hbm.at[src_r], kv_rows, sem1)
        c2 = pltpu.async_copy(q_hbm.at[dst_r], q_rows, sem2)
        c1.wait()
        c2.wait()
        plsc.parallel_loop(0, B // 16)(_group)
        pltpu.sync_copy(out_rows, acc.at[dst_r], add=True)
        return carry

    lax.fori_loop(0, NB, _batch, 0)

    plsc.subcore_barrier()
    pltpu.sync_copy(acc.at[pl.ds(sid * RPT, RPT)],
                    partial_hbm.at[cid, pl.ds(sid * RPT, RPT)])


def _edge_pass(q, kv, idx_packed):
    mesh = plsc.VectorSubcoreMesh(core_axis_name="c", subcore_axis_name="s",
                                  num_cores=NC, num_subcores=NS)
    f = pl.kernel(
        _edge_body,
        out_type=jax.ShapeDtypeStruct((NC, N_ACC, ROW), jnp.float32),
        mesh=mesh,
        scratch_types=[
            pltpu.VMEM((2, B), jnp.int32),
            pltpu.VMEM((B, 2 * HD), jnp.float32),
            pltpu.VMEM((B, HD), jnp.float32),
            pltpu.VMEM((B, ROW), jnp.float32),
            pltpu.VMEM_SHARED((N_ACC, ROW), jnp.float32),
            pltpu.SemaphoreType.DMA,
            pltpu.SemaphoreType.DMA,
        ],
        compiler_params=pltpu.CompilerParams(use_tc_tiling_on_sc=False,
                                             needs_layout_passes=False),
    )
    return f(q, kv, idx_packed)


def _final_body(p_ref, o_ref):
    s = p_ref[0] + p_ref[1]
    den = s[:, HD:HD + H]
    r = jnp.where(den > 0.0, 1.0 / den, 0.0)
    for h in range(H):
        o_ref[:, h * D:(h + 1) * D] = s[:, h * D:(h + 1) * D] * r[:, h:h + 1]


def _finalize(partial):
    blk = 1000
    return pl.pallas_call(
        _final_body,
        grid=(N // blk,),
        in_specs=[pl.BlockSpec((NC, blk, ROW), lambda i: (0, i, 0))],
        out_specs=pl.BlockSpec((blk, HD), lambda i: (i, 0)),
        out_shape=jax.ShapeDtypeStruct((N, HD), jnp.float32),
    )(partial)


def kernel(h, edge_index, Wq, Wk, Wv):
    wkv = jnp.concatenate([Wk, Wv], axis=1)
    q, kv = _project(h, Wq, wkv)
    idx_packed = edge_index.reshape(2, E // B, B).transpose(1, 0, 2)
    partial = _edge_pass(q, kv, idx_packed)
    return _finalize(partial)
